# baseline (speedup 1.0000x reference)
import functools

import jax
import jax.numpy as jnp
from jax import lax
from jax.experimental import pallas as pl
from jax.experimental.pallas import tpu as pltpu, tpu_sc as plsc

N_EDGES = 320000
DIM_EMB = 128
DIM_DICT_ROWS = 400

_G = 80
_GPS = 5
_SC_ROWS = _G * _GPS
_NSC = N_EDGES // _SC_ROWS


def _make_kernel(n_workers: int):
    spw = _NSC // n_workers
    gpw = spw * _GPS
    mesh = plsc.VectorSubcoreMesh(core_axis_name="c", subcore_axis_name="s")

    @functools.partial(
        pl.kernel,
        mesh=mesh,
        out_type=jax.ShapeDtypeStruct((_NSC, _SC_ROWS, DIM_EMB), jnp.float32),
        scratch_types=[
            pltpu.VMEM((gpw, _G), jnp.int32),
            pltpu.VMEM_SHARED((DIM_DICT_ROWS, DIM_EMB), jnp.float32),
            pltpu.VMEM((2 * _SC_ROWS, DIM_EMB), jnp.float32),
            pltpu.SemaphoreType.DMA,
            pltpu.SemaphoreType.DMA,
            pltpu.SemaphoreType.DMA,
            pltpu.SemaphoreType.DMA,
        ],
    )
    def k(et_hbm, table_hbm, out_hbm, idx_all, table_sh, rows, g0, g1, w0, w1):
        gsems, wsems = (g0, g1), (w0, w1)
        wid = lax.axis_index("s") * 2 + lax.axis_index("c")
        base = wid * spw

        pltpu.sync_copy(et_hbm.at[wid], idx_all)

        @pl.when(lax.axis_index("s") == 0)
        def _stage_table():
            pltpu.sync_copy(table_hbm, table_sh)

        plsc.subcore_barrier()

        def gathers(c, b):
            for j in range(_GPS):
                pltpu.async_copy(
                    table_sh.at[idx_all.at[c * _GPS + j]],
                    rows.at[pl.ds(b * _SC_ROWS + j * _G, _G)],
                    gsems[b])

        def wait_gathers(b):
            pltpu.make_async_copy(
                out_hbm.at[0], rows.at[pl.ds(b * _SC_ROWS, _SC_ROWS)],
                gsems[b]).wait()

        def write(c, b):
            pltpu.async_copy(
                rows.at[pl.ds(b * _SC_ROWS, _SC_ROWS)], out_hbm.at[base + c],
                wsems[b])

        def wait_write(c, b):
            pltpu.make_async_copy(
                rows.at[pl.ds(b * _SC_ROWS, _SC_ROWS)], out_hbm.at[base + c],
                wsems[b]).wait()

        gathers(0, 0)
        wait_gathers(0)
        write(0, 0)
        gathers(1, 1)

        def step(c, b):
            wait_gathers(b)
            write(c, b)
            wait_write(c - 1, 1 - b)
            gathers(c + 1, 1 - b)

        @pl.loop(0, (spw - 2) // 2)
        def grp(g):
            step(2 * g + 1, 1)
            step(2 * g + 2, 0)

        c = spw - 2
        step(c, 1)
        c = spw - 1
        wait_gathers(0)
        write(c, 0)
        wait_write(c - 1, 1)
        wait_write(c, 0)

    return k


def kernel(edge_type, embedding):
    et = edge_type.astype(jnp.int32).reshape(32, _NSC // 32 * _GPS, _G)
    out = _make_kernel(32)(et, embedding)
    return out.reshape(N_EDGES, DIM_EMB)
